# baseline (speedup 1.0000x reference)
import jax
import jax.numpy as jnp
from jax import lax
from jax.experimental import pallas as pl
from jax.experimental.pallas import tpu as pltpu
from jax.experimental.pallas import tpu_sc as plsc

NUM_EMB = 1_000_000
DIM = 64
BATCH = 4096
SEQ = 200

NC = 2
NS = 16
NW = NC * NS

NBLK_A = 7813
NBLK3 = (NBLK_A + 2) // 3
TBL_PAD = NBLK_A * 128
BT = BATCH // 128


def _transpose_body(embt_hbm, tlin_hbm, in_v, out_v, isem, osem):
    wid = lax.axis_index("s") * NC + lax.axis_index("c")
    k16 = lax.iota(jnp.int32, 16)
    nj = (NBLK3 - wid + NW - 1) // NW

    def col0(i):
        return lax.min((wid + i * NW) * 3, NBLK_A - 3)

    def fire_in(i, slot):
        j = pl.multiple_of(col0(i) * 128, 128)
        for dt in range(8):
            pltpu.async_copy(embt_hbm.at[pl.ds(dt * 8, 8), pl.ds(j, 384)],
                             in_v.at[slot, pl.ds(dt * 8, 8)], isem.at[slot])

    def wait_in(i, slot):
        j = pl.multiple_of(col0(i) * 128, 128)
        for dt in range(8):
            pltpu.make_async_copy(embt_hbm.at[pl.ds(dt * 8, 8), pl.ds(j, 384)],
                                  in_v.at[slot, pl.ds(dt * 8, 8)], isem.at[slot]).wait()

    def fire_out(i, slot):
        pltpu.async_copy(out_v.at[slot],
                         tlin_hbm.at[pl.ds(col0(i) * 128 * DIM, 384 * DIM)],
                         osem.at[slot])

    def wait_out(i, slot):
        pltpu.make_async_copy(out_v.at[slot],
                              tlin_hbm.at[pl.ds(col0(i) * 128 * DIM, 384 * DIM)],
                              osem.at[slot]).wait()

    din_c = [[d0 + (k16 + s) % 16 for s in range(16)] for d0 in (0, 16, 32, 48)]
    oix_c = [[k16 * DIM + d0 + (k16 + s) % 16 for s in range(16)]
             for d0 in (0, 16, 32, 48)]

    fire_in(0, 0)

    def step(i, carry):
        slot = lax.rem(i, 2)
        sv = jnp.full((16,), slot, jnp.int32)

        @pl.when(i + 1 < nj)
        def _():
            fire_in(i + 1, 1 - slot)

        wait_in(i, slot)

        @pl.when(i >= 2)
        def _():
            wait_out(i - 2, slot)

        def shuf(rb, c):
            rvec = rb * 16 + k16
            r064 = rb * (16 * DIM)
            for d0 in range(4):
                for s in range(16):
                    v = plsc.load_gather(in_v, [sv, din_c[d0][s], rvec])
                    plsc.store_scatter(out_v, [sv, oix_c[d0][s] + r064], v)
            return c

        lax.fori_loop(0, 24, shuf, 0)
        fire_out(i, slot)
        return carry

    lax.fori_loop(0, nj, step, 0)

    def drain(q, carry):
        i = nj - 2 + q

        @pl.when(i >= 0)
        def _():
            wait_out(i, lax.rem(i, 2))
        return carry

    lax.fori_loop(0, 2, drain, 0)


def _gather_body(ids_hbm, table_hbm, out_hbm, idx_v, g_v, t_v, xsem, gsem, ssem):
    wid = lax.axis_index("s") * NC + lax.axis_index("c")
    k16 = lax.iota(jnp.int32, 16)

    pltpu.async_copy(ids_hbm.at[pl.ds(0, SEQ), pl.ds(wid * 128, 128)],
                     idx_v, xsem)
    pltpu.make_async_copy(ids_hbm.at[pl.ds(0, SEQ), pl.ds(wid * 128, 128)],
                          idx_v, xsem).wait()

    def fire_g(s, slot):
        pltpu.async_copy(table_hbm.at[idx_v.at[s]], g_v.at[slot], gsem.at[slot])

    def wait_g(s, slot):
        pltpu.make_async_copy(table_hbm.at[idx_v.at[s]], g_v.at[slot],
                              gsem.at[slot]).wait()

    def fire_s(s, slot):
        for dt in range(8):
            pltpu.async_copy(t_v.at[slot, pl.ds(dt * 8, 8)], out_hbm.at[s, dt, wid],
                             ssem.at[slot])

    def wait_s(s, slot):
        for dt in range(8):
            pltpu.make_async_copy(t_v.at[slot, pl.ds(dt * 8, 8)], out_hbm.at[s, dt, wid],
                                  ssem.at[slot]).wait()

    sconst = [jnp.full((16,), b, jnp.int32) for b in range(4)]
    zconst = jnp.full((16,), 0, jnp.int32)
    dvec_c = [[d0 + (k16 + s) % 16 for s in range(16)] for d0 in (0, 16, 32, 48)]
    dt_c = [[v // 8 for v in row] for row in dvec_c]
    di_c = [[v % 8 for v in row] for row in dvec_c]

    def shuffle(slot):
        def shuf(bb, c):
            bivec = bb * 16 + k16
            for d0 in range(4):
                for s in range(16):
                    v = plsc.load_gather(g_v, [sconst[slot], bivec, dvec_c[d0][s]])
                    plsc.store_scatter(
                        t_v, [sconst[slot], dvec_c[d0][s], bivec], v)
            return c
        lax.fori_loop(0, 8, shuf, 0)

    for pb in range(3):
        fire_g(pb, pb)
    ns4 = SEQ // 4

    def step(s4, carry):
        for b in range(4):
            s = 4 * s4 + b

            @pl.when(s + 3 < SEQ)
            def _():
                fire_g(s + 3, (b + 3) % 4)

            wait_g(s, b)

            @pl.when(s >= 4)
            def _():
                wait_s(s - 4, b)

            shuffle(b)
            fire_s(s, b)
        return carry

    lax.fori_loop(0, ns4, step, 0)
    for q in range(4):
        wait_s(SEQ - 4 + q, q % 4)


def _make_transpose_kernel():
    mesh = plsc.VectorSubcoreMesh(core_axis_name="c", subcore_axis_name="s")
    return pl.kernel(
        _transpose_body,
        out_type=jax.ShapeDtypeStruct((TBL_PAD * DIM,), jnp.float32),
        mesh=mesh,
        scratch_types=[
            pltpu.VMEM((2, DIM, 384), jnp.float32),
            pltpu.VMEM((2, 384 * DIM), jnp.float32),
            pltpu.SemaphoreType.DMA((2,)),
            pltpu.SemaphoreType.DMA((2,)),
        ],
        compiler_params=pltpu.CompilerParams(
            use_tc_tiling_on_sc=True, needs_layout_passes=False,
            disable_bounds_checks=True),
    )


def _make_gather_kernel():
    mesh = plsc.VectorSubcoreMesh(core_axis_name="c", subcore_axis_name="s")
    return pl.kernel(
        _gather_body,
        out_type=jax.ShapeDtypeStruct((SEQ, 8, BT, 8, 128), jnp.float32),
        mesh=mesh,
        scratch_types=[
            pltpu.VMEM((SEQ, 128), jnp.int32),
            pltpu.VMEM((4, 128, DIM), jnp.float32),
            pltpu.VMEM((4, DIM, 128), jnp.float32),
            pltpu.SemaphoreType.DMA,
            pltpu.SemaphoreType.DMA((4,)),
            pltpu.SemaphoreType.DMA((4,)),
        ],
        compiler_params=pltpu.CompilerParams(
            use_tc_tiling_on_sc=False, needs_layout_passes=False),
    )


@jax.jit
def _emb_call(ids_t, embt):
    tlin = _make_transpose_kernel()(embt)
    table = tlin.reshape(TBL_PAD, DIM)
    return _make_gather_kernel()(ids_t, table)


def kernel(token_ids, embedding):
    embt = embedding.T
    ids_t = token_ids.T.astype(jnp.int32)
    out5d = _emb_call(ids_t, embt)
    return out5d.transpose(2, 4, 0, 1, 3).reshape(BATCH, SEQ, DIM)


# speedup vs baseline: 1.4207x; 1.4207x over previous
import functools

import jax
import jax.numpy as jnp
from jax import lax
from jax.experimental import pallas as pl
from jax.experimental.pallas import tpu as pltpu
from jax.experimental.pallas import tpu_sc as plsc

NUM_EMB = 1_000_000
DIM = 64
BATCH = 4096
SEQ = 200
NTOK = BATCH * SEQ

NC = 2
NS = 16
NW = NC * NS
PER_W = NTOK // NW
CHUNK = 800
NBUF = 2
NCHUNK = PER_W // CHUNK


def _emb_body(ids_hbm, table_hbm, out_hbm, idx_v, rows_v, gsem, ssem):
    wid = lax.axis_index("s") * NC + lax.axis_index("c")
    base = wid * PER_W

    pltpu.sync_copy(ids_hbm.at[pl.ds(base, PER_W)], idx_v)

    def gather(i, b):
        pltpu.async_copy(
            table_hbm.at[idx_v.at[pl.ds(i * CHUNK, CHUNK)]],
            rows_v.at[b], gsem.at[b])

    def store(i, b):
        pltpu.async_copy(
            rows_v.at[b], out_hbm.at[pl.ds(base + i * CHUNK, CHUNK)],
            ssem.at[b])

    def wait_gather(i, b):
        pltpu.make_async_copy(
            table_hbm.at[idx_v.at[pl.ds(i * CHUNK, CHUNK)]],
            rows_v.at[b], gsem.at[b]).wait()

    def wait_store(i, b):
        pltpu.make_async_copy(
            rows_v.at[b], out_hbm.at[pl.ds(base + i * CHUNK, CHUNK)],
            ssem.at[b]).wait()

    gather(0, 0)

    def step(i, carry):
        b = lax.rem(i, NBUF)
        nb = lax.rem(i + 1, NBUF)

        @pl.when(i + 1 < NCHUNK)
        def _():
            @pl.when(i + 1 >= NBUF)
            def _():
                wait_store(i + 1 - NBUF, nb)
            gather(i + 1, nb)

        wait_gather(i, b)
        store(i, b)
        return carry

    lax.fori_loop(0, NCHUNK, step, 0)
    wait_store(NCHUNK - NBUF, lax.rem(NCHUNK - NBUF, NBUF))
    wait_store(NCHUNK - 1, lax.rem(NCHUNK - 1, NBUF))


@jax.jit
def _emb_call(ids_flat, table):
    grid_kernel = pl.kernel(
        _emb_body,
        out_type=jax.ShapeDtypeStruct((NTOK, DIM), jnp.float32),
        mesh=plsc.VectorSubcoreMesh(core_axis_name="c", subcore_axis_name="s"),
        scratch_types=[
            pltpu.VMEM((PER_W,), jnp.int32),
            pltpu.VMEM((NBUF, CHUNK, DIM), jnp.float32),
            pltpu.SemaphoreType.DMA((NBUF,)),
            pltpu.SemaphoreType.DMA((NBUF,)),
        ],
        compiler_params=pltpu.CompilerParams(use_tc_tiling_on_sc=False),
    )
    return grid_kernel(ids_flat, table)


def kernel(token_ids, embedding):
    ids_flat = token_ids.reshape(NTOK).astype(jnp.int32)
    out = _emb_call(ids_flat, embedding)
    return out.reshape(BATCH, SEQ, DIM)
